# baseline (speedup 1.0000x reference)
import functools

import jax
import jax.numpy as jnp
from jax import lax
from jax.experimental import pallas as pl
from jax.experimental.pallas import tpu as pltpu
from jax.experimental.pallas import tpu_sc as plsc

_NC = 2
_NS = 16
_NW = _NC * _NS
_B = 128


def _k1_body(x_ref, w_ref, a_ref, p_ref):
    v = lax.dot_general(a_ref[...], w_ref[...], (((1,), (0,)), ((), ())),
                        preferred_element_type=jnp.float32)
    p_ref[...] = lax.dot_general(x_ref[...], v, (((1,), (1,)), ((), ())),
                                 preferred_element_type=jnp.float32)


def _k2_body(cpw, src_ref, dst_ref, p_ref, s_ref,
             sidx, didx, ps, pd, sbuf, sem):
    wid = lax.axis_index("s") * _NC + lax.axis_index("c")

    def chunk(i, carry):
        base = wid * (cpw * _B) + i * _B
        pltpu.sync_copy(src_ref.at[pl.ds(base, _B)], sidx)
        pltpu.sync_copy(dst_ref.at[pl.ds(base, _B)], didx)
        pltpu.async_copy(p_ref.at[sidx], ps, sem).wait()
        pltpu.async_copy(p_ref.at[didx], pd, sem).wait()

        def vec(j, c):
            t = ps[pl.ds(j * 16, 16)] + pd[pl.ds(j * 16, 16)]
            sbuf[pl.ds(j * 16, 16)] = jnp.maximum(t, 0.2 * t)
            return c

        lax.fori_loop(0, _B // 16, vec, 0)
        pltpu.sync_copy(sbuf, s_ref.at[pl.ds(base, _B)])
        return carry

    lax.fori_loop(0, cpw, chunk, 0)


def _k3_body(nrows, s_ref, w_ref):
    sv = s_ref[...]
    rows = lax.broadcasted_iota(jnp.int32, sv.shape, 0)
    mask = rows < nrows
    m = jnp.max(jnp.where(mask, sv, -jnp.inf))
    e = jnp.where(mask, jnp.exp(sv - m), 0.0)
    w_ref[...] = e / jnp.sum(e)


def _vsqrt(d2):
    ii = plsc.bitcast(d2, jnp.int32)
    g = plsc.bitcast((ii >> 1) + 0x1FBD1DF5, jnp.float32)
    for _ in range(3):
        g = 0.5 * (g + d2 / g)
    return jnp.where(d2 > 0.0, g, 0.0)


def _k4_body(n, d, cpw, src_ref, dst_ref, wgt_ref, x_ref, out_ref,
             sidx, didx, wbuf, xs, xd, msg, acc, sem, sem2):
    cid = lax.axis_index("c")
    sid = lax.axis_index("s")
    wid = sid * _NC + cid
    rows_per_sub = (n // _NS) // 8 * 8
    zrows = 104
    rbase = pl.multiple_of(sid * rows_per_sub, 8)
    tail = n - rows_per_sub * _NS

    def zrow(i, c):
        for k8 in range(d // 16):
            msg[i, pl.ds(k8 * 16, 16)] = jnp.zeros((16,), jnp.float32)
        return c

    lax.fori_loop(0, zrows, zrow, 0)
    for kblk in range(rows_per_sub // zrows):
        pltpu.sync_copy(msg.at[pl.ds(0, zrows)],
                        acc.at[pl.ds(rbase + kblk * zrows, zrows)])

    @pl.when(sid == _NS - 1)
    def _zero_tail():
        pltpu.sync_copy(msg.at[pl.ds(0, tail)],
                        acc.at[pl.ds(rows_per_sub * _NS, tail)])

    plsc.subcore_barrier()

    lane = lax.iota(jnp.int32, 16)

    def chunk(i, carry):
        base = wid * (cpw * _B) + i * _B
        pltpu.sync_copy(src_ref.at[pl.ds(base, _B)], sidx)
        pltpu.sync_copy(dst_ref.at[pl.ds(base, _B)], didx)
        pltpu.sync_copy(wgt_ref.at[pl.ds(base, _B)], wbuf)
        c1 = pltpu.async_copy(x_ref.at[sidx], xs, sem)
        c2 = pltpu.async_copy(x_ref.at[didx], xd, sem2)
        c1.wait()
        c2.wait()

        def group(g, c):
            eidx = lane + g * 16

            def dk8(k16, a2):
                kb = k16 * 8
                for u in range(8):
                    kk = jnp.full((16,), 0, jnp.int32) + (kb + u)
                    dxy = (plsc.load_gather(xs, [eidx, kk])
                           - plsc.load_gather(xd, [eidx, kk]))
                    a2 = a2 + dxy * dxy
                return a2

            d2 = lax.fori_loop(0, d // 8, dk8,
                               jnp.zeros((16,), jnp.float32))
            cf = wbuf[pl.ds(g * 16, 16)] * _vsqrt(d2)

            def mk8(k16, cc):
                kb = k16 * 8
                for u in range(8):
                    kk = jnp.full((16,), 0, jnp.int32) + (kb + u)
                    plsc.store_scatter(
                        msg, [eidx, kk],
                        plsc.load_gather(xd, [eidx, kk]) * cf)
                return cc

            lax.fori_loop(0, d // 8, mk8, 0)
            return c

        lax.fori_loop(0, _B // 16, group, 0)
        pltpu.sync_copy(msg, acc.at[sidx], add=True)
        return carry

    lax.fori_loop(0, cpw, chunk, 0)
    plsc.subcore_barrier()

    pltpu.sync_copy(acc.at[pl.ds(rbase, rows_per_sub)],
                    out_ref.at[cid, pl.ds(rbase, rows_per_sub)])

    @pl.when(sid == _NS - 1)
    def _dump_tail():
        pltpu.sync_copy(acc.at[pl.ds(rows_per_sub * _NS, tail)],
                        out_ref.at[cid, pl.ds(rows_per_sub * _NS, tail)])


def _k5_body(a_ref, b_ref, o_ref):
    o_ref[...] = a_ref[...] + b_ref[...]


def kernel(x, edge_index, W, a):
    n, d = x.shape
    e = edge_index.shape[1]
    cpw = -(-e // (_NW * _B))
    epad = cpw * _NW * _B
    mesh = plsc.VectorSubcoreMesh(core_axis_name="c", subcore_axis_name="s")

    src = jnp.pad(edge_index[0], (0, epad - e))
    dst = jnp.pad(edge_index[1], (0, epad - e))

    p = pl.pallas_call(
        _k1_body,
        out_shape=jax.ShapeDtypeStruct((n, 1), jnp.float32),
    )(x, W, a.reshape(1, d)).reshape(n)

    k2 = pl.kernel(
        functools.partial(_k2_body, cpw),
        out_type=jax.ShapeDtypeStruct((epad,), jnp.float32),
        mesh=mesh,
        scratch_types=[
            pltpu.VMEM((_B,), jnp.int32),
            pltpu.VMEM((_B,), jnp.int32),
            pltpu.VMEM((_B,), jnp.float32),
            pltpu.VMEM((_B,), jnp.float32),
            pltpu.VMEM((_B,), jnp.float32),
            pltpu.SemaphoreType.DMA,
        ],
    )
    s = k2(src, dst, p)

    w = pl.pallas_call(
        functools.partial(_k3_body, e // 128),
        out_shape=jax.ShapeDtypeStruct((epad // 128, 128), jnp.float32),
    )(s.reshape(epad // 128, 128)).reshape(epad)

    k4 = pl.kernel(
        functools.partial(_k4_body, n, d, cpw),
        out_type=jax.ShapeDtypeStruct((_NC, n, d), jnp.float32),
        mesh=mesh,
        compiler_params=pltpu.CompilerParams(needs_layout_passes=False),
        scratch_types=[
            pltpu.VMEM((_B,), jnp.int32),
            pltpu.VMEM((_B,), jnp.int32),
            pltpu.VMEM((_B,), jnp.float32),
            pltpu.VMEM((_B, d), jnp.float32),
            pltpu.VMEM((_B, d), jnp.float32),
            pltpu.VMEM((_B, d), jnp.float32),
            pltpu.VMEM_SHARED((n, d), jnp.float32),
            pltpu.SemaphoreType.DMA,
            pltpu.SemaphoreType.DMA,
        ],
    )
    parts = k4(src, dst, w, x)

    return pl.pallas_call(
        _k5_body,
        out_shape=jax.ShapeDtypeStruct((n, d), jnp.float32),
    )(parts[0], parts[1])


# speedup vs baseline: 1.2049x; 1.2049x over previous
import functools

import jax
import jax.numpy as jnp
from jax import lax
from jax.experimental import pallas as pl
from jax.experimental.pallas import tpu as pltpu
from jax.experimental.pallas import tpu_sc as plsc

_NC = 2
_NS = 16
_NW = _NC * _NS
_B = 128


def _k1_body(x_ref, w_ref, a_ref, p_ref):
    v = lax.dot_general(a_ref[...], w_ref[...], (((1,), (0,)), ((), ())),
                        preferred_element_type=jnp.float32)
    p_ref[...] = lax.dot_general(x_ref[...], v, (((1,), (1,)), ((), ())),
                                 preferred_element_type=jnp.float32)


def _k2_body(cpw, src_ref, dst_ref, p_ref, s_ref,
             sidx, didx, ps, pd, sbuf, sem):
    wid = lax.axis_index("s") * _NC + lax.axis_index("c")

    def chunk(i, carry):
        base = wid * (cpw * _B) + i * _B
        pltpu.sync_copy(src_ref.at[pl.ds(base, _B)], sidx)
        pltpu.sync_copy(dst_ref.at[pl.ds(base, _B)], didx)
        pltpu.async_copy(p_ref.at[sidx], ps, sem).wait()
        pltpu.async_copy(p_ref.at[didx], pd, sem).wait()

        def vec(j, c):
            t = ps[pl.ds(j * 16, 16)] + pd[pl.ds(j * 16, 16)]
            sbuf[pl.ds(j * 16, 16)] = jnp.maximum(t, 0.2 * t)
            return c

        lax.fori_loop(0, _B // 16, vec, 0)
        pltpu.sync_copy(sbuf, s_ref.at[pl.ds(base, _B)])
        return carry

    lax.fori_loop(0, cpw, chunk, 0)


def _k3_body(nrows, s_ref, w_ref):
    sv = s_ref[...]
    rows = lax.broadcasted_iota(jnp.int32, sv.shape, 0)
    mask = rows < nrows
    m = jnp.max(jnp.where(mask, sv, -jnp.inf))
    e = jnp.where(mask, jnp.exp(sv - m), 0.0)
    w_ref[...] = e / jnp.sum(e)


def _vsqrt(d2):
    ii = plsc.bitcast(d2, jnp.int32)
    g = plsc.bitcast((ii >> 1) + 0x1FBD1DF5, jnp.float32)
    for _ in range(3):
        g = 0.5 * (g + d2 / g)
    return jnp.where(d2 > 0.0, g, 0.0)


def _k4_body(n, d, cpw, src_ref, dst_ref, wgt_ref, x_ref, out_ref,
             sidx, didx, wbuf, xs, xd, msg, acc, sem, sem2):
    cid = lax.axis_index("c")
    sid = lax.axis_index("s")
    wid = sid * _NC + cid
    rows_per_sub = (n // _NS) // 8 * 8
    zrows = 104
    rbase = pl.multiple_of(sid * rows_per_sub, 8)
    tail = n - rows_per_sub * _NS

    def zrow(i, c):
        for k8 in range(d // 16):
            msg[i, pl.ds(k8 * 16, 16)] = jnp.zeros((16,), jnp.float32)
        return c

    lax.fori_loop(0, zrows, zrow, 0)
    for kblk in range(rows_per_sub // zrows):
        pltpu.sync_copy(msg.at[pl.ds(0, zrows)],
                        acc.at[pl.ds(rbase + kblk * zrows, zrows)])

    @pl.when(sid == _NS - 1)
    def _zero_tail():
        pltpu.sync_copy(msg.at[pl.ds(0, tail)],
                        acc.at[pl.ds(rows_per_sub * _NS, tail)])

    plsc.subcore_barrier()

    lane = lax.iota(jnp.int32, 16)

    def chunk(i, carry):
        base = wid * (cpw * _B) + i * _B
        pltpu.sync_copy(src_ref.at[pl.ds(base, _B)], sidx)
        pltpu.sync_copy(dst_ref.at[pl.ds(base, _B)], didx)
        pltpu.sync_copy(wgt_ref.at[pl.ds(base, _B)], wbuf)
        c1 = pltpu.async_copy(x_ref.at[sidx], xs, sem)
        c2 = pltpu.async_copy(x_ref.at[didx], xd, sem2)
        c1.wait()
        c2.wait()

        def group(g, c):
            eidx = lane + g * 16

            @plsc.parallel_loop(0, d, 1, unroll=8,
                                carry=jnp.zeros((16,), jnp.float32))
            def d2(k, a2):
                kk = jnp.full((16,), 0, jnp.int32) + k
                dxy = (plsc.load_gather(xs, [eidx, kk])
                       - plsc.load_gather(xd, [eidx, kk]))
                return a2 + dxy * dxy

            cf = wbuf[pl.ds(g * 16, 16)] * _vsqrt(d2)

            @plsc.parallel_loop(0, d, 1, unroll=8)
            def _mk(k):
                kk = jnp.full((16,), 0, jnp.int32) + k
                plsc.store_scatter(
                    msg, [eidx, kk],
                    plsc.load_gather(xd, [eidx, kk]) * cf)

            return c

        lax.fori_loop(0, _B // 16, group, 0)
        pltpu.sync_copy(msg, acc.at[sidx], add=True)
        return carry

    lax.fori_loop(0, cpw, chunk, 0)
    plsc.subcore_barrier()

    pltpu.sync_copy(acc.at[pl.ds(rbase, rows_per_sub)],
                    out_ref.at[cid, pl.ds(rbase, rows_per_sub)])

    @pl.when(sid == _NS - 1)
    def _dump_tail():
        pltpu.sync_copy(acc.at[pl.ds(rows_per_sub * _NS, tail)],
                        out_ref.at[cid, pl.ds(rows_per_sub * _NS, tail)])


def _k5_body(a_ref, b_ref, o_ref):
    o_ref[...] = a_ref[...] + b_ref[...]


def kernel(x, edge_index, W, a):
    n, d = x.shape
    e = edge_index.shape[1]
    cpw = -(-e // (_NW * _B))
    epad = cpw * _NW * _B
    mesh = plsc.VectorSubcoreMesh(core_axis_name="c", subcore_axis_name="s")

    src = jnp.pad(edge_index[0], (0, epad - e))
    dst = jnp.pad(edge_index[1], (0, epad - e))

    p = pl.pallas_call(
        _k1_body,
        out_shape=jax.ShapeDtypeStruct((n, 1), jnp.float32),
    )(x, W, a.reshape(1, d)).reshape(n)

    k2 = pl.kernel(
        functools.partial(_k2_body, cpw),
        out_type=jax.ShapeDtypeStruct((epad,), jnp.float32),
        mesh=mesh,
        scratch_types=[
            pltpu.VMEM((_B,), jnp.int32),
            pltpu.VMEM((_B,), jnp.int32),
            pltpu.VMEM((_B,), jnp.float32),
            pltpu.VMEM((_B,), jnp.float32),
            pltpu.VMEM((_B,), jnp.float32),
            pltpu.SemaphoreType.DMA,
        ],
    )
    s = k2(src, dst, p)

    w = pl.pallas_call(
        functools.partial(_k3_body, e // 128),
        out_shape=jax.ShapeDtypeStruct((epad // 128, 128), jnp.float32),
    )(s.reshape(epad // 128, 128)).reshape(epad)

    k4 = pl.kernel(
        functools.partial(_k4_body, n, d, cpw),
        out_type=jax.ShapeDtypeStruct((_NC, n, d), jnp.float32),
        mesh=mesh,
        compiler_params=pltpu.CompilerParams(needs_layout_passes=False),
        scratch_types=[
            pltpu.VMEM((_B,), jnp.int32),
            pltpu.VMEM((_B,), jnp.int32),
            pltpu.VMEM((_B,), jnp.float32),
            pltpu.VMEM((_B, d), jnp.float32),
            pltpu.VMEM((_B, d), jnp.float32),
            pltpu.VMEM((_B, d), jnp.float32),
            pltpu.VMEM_SHARED((n, d), jnp.float32),
            pltpu.SemaphoreType.DMA,
            pltpu.SemaphoreType.DMA,
        ],
    )
    parts = k4(src, dst, w, x)

    return pl.pallas_call(
        _k5_body,
        out_shape=jax.ShapeDtypeStruct((n, d), jnp.float32),
    )(parts[0], parts[1])


# speedup vs baseline: 3.1863x; 2.6446x over previous
import functools

import jax
import jax.numpy as jnp
from jax import lax
from jax.experimental import pallas as pl
from jax.experimental.pallas import tpu as pltpu
from jax.experimental.pallas import tpu_sc as plsc

_NC = 2
_NS = 16
_NW = _NC * _NS
_B = 128


def _k1_body(x_ref, w_ref, a_ref, p_ref):
    v = lax.dot_general(a_ref[...], w_ref[...], (((1,), (0,)), ((), ())),
                        preferred_element_type=jnp.float32)
    p_ref[...] = lax.dot_general(x_ref[...], v, (((1,), (1,)), ((), ())),
                                 preferred_element_type=jnp.float32)


def _k2_body(cpw, src_ref, dst_ref, p_ref, s_ref,
             sidx, didx, ps, pd, sbuf, sem):
    wid = lax.axis_index("s") * _NC + lax.axis_index("c")

    def chunk(i, carry):
        base = wid * (cpw * _B) + i * _B
        pltpu.sync_copy(src_ref.at[pl.ds(base, _B)], sidx)
        pltpu.sync_copy(dst_ref.at[pl.ds(base, _B)], didx)
        pltpu.async_copy(p_ref.at[sidx], ps, sem).wait()
        pltpu.async_copy(p_ref.at[didx], pd, sem).wait()

        def vec(j, c):
            t = ps[pl.ds(j * 16, 16)] + pd[pl.ds(j * 16, 16)]
            sbuf[pl.ds(j * 16, 16)] = jnp.maximum(t, 0.2 * t)
            return c

        lax.fori_loop(0, _B // 16, vec, 0)
        pltpu.sync_copy(sbuf, s_ref.at[pl.ds(base, _B)])
        return carry

    lax.fori_loop(0, cpw, chunk, 0)


def _k3_body(nrows, s_ref, w_ref):
    sv = s_ref[...]
    rows = lax.broadcasted_iota(jnp.int32, sv.shape, 0)
    mask = rows < nrows
    m = jnp.max(jnp.where(mask, sv, -jnp.inf))
    e = jnp.where(mask, jnp.exp(sv - m), 0.0)
    w_ref[...] = e / jnp.sum(e)


def _vsqrt(d2):
    ii = plsc.bitcast(d2, jnp.int32)
    g = plsc.bitcast((ii >> 1) + 0x1FBD1DF5, jnp.float32)
    for _ in range(3):
        g = 0.5 * (g + d2 / g)
    return jnp.where(d2 > 0.0, g, 0.0)


def _k4_body(n, d, cpw, src_ref, dst_ref, wgt_ref, x_ref, out_ref,
             sidx, didx, wbuf, xs, xd, msg, acc, sem, sem2):
    cid = lax.axis_index("c")
    sid = lax.axis_index("s")
    wid = sid * _NC + cid
    rows_per_sub = (n // _NS) // 8 * 8
    zrows = 104
    rbase = pl.multiple_of(sid * rows_per_sub, 8)
    tail = n - rows_per_sub * _NS

    def zrow(i, c):
        for k8 in range(d // 16):
            msg[i, pl.ds(k8 * 16, 16)] = jnp.zeros((16,), jnp.float32)
        return c

    lax.fori_loop(0, zrows, zrow, 0)
    for kblk in range(rows_per_sub // zrows):
        pltpu.sync_copy(msg.at[pl.ds(0, zrows)],
                        acc.at[pl.ds(rbase + kblk * zrows, zrows)])

    @pl.when(sid == _NS - 1)
    def _zero_tail():
        pltpu.sync_copy(msg.at[pl.ds(0, tail)],
                        acc.at[pl.ds(rows_per_sub * _NS, tail)])

    plsc.subcore_barrier()

    lane = lax.iota(jnp.int32, 16)

    def chunk(i, carry):
        base = wid * (cpw * _B) + i * _B
        pltpu.sync_copy(src_ref.at[pl.ds(base, _B)], sidx)
        pltpu.sync_copy(dst_ref.at[pl.ds(base, _B)], didx)
        pltpu.sync_copy(wgt_ref.at[pl.ds(base, _B)], wbuf)
        c1 = pltpu.async_copy(x_ref.at[sidx], xs, sem)
        c2 = pltpu.async_copy(x_ref.at[didx], xd, sem2)
        c1.wait()
        c2.wait()

        def group(g, c):
            g16 = g * 16
            d2v = jnp.zeros((16,), jnp.float32)
            for u in range(16):
                accv = None
                for k8 in range(d // 16):
                    sl = pl.ds(k8 * 16, 16)
                    dv = xs[g16 + u, sl] - xd[g16 + u, sl]
                    accv = dv * dv if accv is None else accv + dv * dv
                b16 = jnp.full((16,), 0.0, jnp.float32) + jnp.sum(accv)
                d2v = jnp.where(lane == u, b16, d2v)
            cf = wbuf[pl.ds(g16, 16)] * _vsqrt(d2v)
            for u in range(16):
                cu = jnp.sum(jnp.where(lane == u, cf, 0.0))
                cb = jnp.full((16,), 0.0, jnp.float32) + cu
                for k8 in range(d // 16):
                    sl = pl.ds(k8 * 16, 16)
                    msg[g16 + u, sl] = xd[g16 + u, sl] * cb
            return c

        lax.fori_loop(0, _B // 16, group, 0)
        pltpu.sync_copy(msg, acc.at[sidx], add=True)
        return carry

    lax.fori_loop(0, cpw, chunk, 0)
    plsc.subcore_barrier()

    pltpu.sync_copy(acc.at[pl.ds(rbase, rows_per_sub)],
                    out_ref.at[cid, pl.ds(rbase, rows_per_sub)])

    @pl.when(sid == _NS - 1)
    def _dump_tail():
        pltpu.sync_copy(acc.at[pl.ds(rows_per_sub * _NS, tail)],
                        out_ref.at[cid, pl.ds(rows_per_sub * _NS, tail)])


def _k5_body(a_ref, b_ref, o_ref):
    o_ref[...] = a_ref[...] + b_ref[...]


def kernel(x, edge_index, W, a):
    n, d = x.shape
    e = edge_index.shape[1]
    cpw = -(-e // (_NW * _B))
    epad = cpw * _NW * _B
    mesh = plsc.VectorSubcoreMesh(core_axis_name="c", subcore_axis_name="s")

    src = jnp.pad(edge_index[0], (0, epad - e))
    dst = jnp.pad(edge_index[1], (0, epad - e))

    p = pl.pallas_call(
        _k1_body,
        out_shape=jax.ShapeDtypeStruct((n, 1), jnp.float32),
    )(x, W, a.reshape(1, d)).reshape(n)

    k2 = pl.kernel(
        functools.partial(_k2_body, cpw),
        out_type=jax.ShapeDtypeStruct((epad,), jnp.float32),
        mesh=mesh,
        scratch_types=[
            pltpu.VMEM((_B,), jnp.int32),
            pltpu.VMEM((_B,), jnp.int32),
            pltpu.VMEM((_B,), jnp.float32),
            pltpu.VMEM((_B,), jnp.float32),
            pltpu.VMEM((_B,), jnp.float32),
            pltpu.SemaphoreType.DMA,
        ],
    )
    s = k2(src, dst, p)

    w = pl.pallas_call(
        functools.partial(_k3_body, e // 128),
        out_shape=jax.ShapeDtypeStruct((epad // 128, 128), jnp.float32),
    )(s.reshape(epad // 128, 128)).reshape(epad)

    k4 = pl.kernel(
        functools.partial(_k4_body, n, d, cpw),
        out_type=jax.ShapeDtypeStruct((_NC, n, d), jnp.float32),
        mesh=mesh,
        compiler_params=pltpu.CompilerParams(needs_layout_passes=False),
        scratch_types=[
            pltpu.VMEM((_B,), jnp.int32),
            pltpu.VMEM((_B,), jnp.int32),
            pltpu.VMEM((_B,), jnp.float32),
            pltpu.VMEM((_B, d), jnp.float32),
            pltpu.VMEM((_B, d), jnp.float32),
            pltpu.VMEM((_B, d), jnp.float32),
            pltpu.VMEM_SHARED((n, d), jnp.float32),
            pltpu.SemaphoreType.DMA,
            pltpu.SemaphoreType.DMA,
        ],
    )
    parts = k4(src, dst, w, x)

    return pl.pallas_call(
        _k5_body,
        out_shape=jax.ShapeDtypeStruct((n, d), jnp.float32),
    )(parts[0], parts[1])
